# baseline (speedup 1.0000x reference)
import functools
import math

import jax
import jax.numpy as jnp
from jax import lax
from jax.experimental import pallas as pl
from jax.experimental.pallas import tpu as pltpu
from jax.experimental.pallas import tpu_sc as plsc

H = 128
NRBF = 32
SW = 144
NC, NS, LANES = 2, 16, 16
NW = NC * NS
C = 16


def _proj3_body(x_ref, wa_ref, wb_ref, wc_ref, oa_ref, ob_ref, oc_ref):
    xv = x_ref[...]
    oa_ref[...] = jnp.dot(xv, wa_ref[...], preferred_element_type=jnp.float32)
    ob_ref[...] = jnp.dot(xv, wb_ref[...], preferred_element_type=jnp.float32)
    oc_ref[...] = jnp.dot(xv, wc_ref[...], preferred_element_type=jnp.float32)


def _count_col(rows):
    lane = lax.broadcasted_iota(jnp.int32, (rows, SW - H), 1)
    return jnp.where(lane == 0, 1.0, 0.0).astype(jnp.float32)


def _eproj_body(a_ref, w_ref, b_ref, o_ref):
    o_ref[:, :H] = (jnp.dot(a_ref[...], w_ref[...],
                            preferred_element_type=jnp.float32) + b_ref[...])
    o_ref[:, H:] = _count_col(o_ref.shape[0])


def _tproj_body(ang_ref, cen_ref, w_ref, b_ref, o_ref, *, inv_sig2):
    d = ang_ref[...] - cen_ref[...]
    rbf = jnp.exp(-(d * d) * inv_sig2)
    o_ref[:, :H] = (jnp.dot(rbf, w_ref[...],
                            preferred_element_type=jnp.float32) + b_ref[...])
    o_ref[:, H:] = _count_col(o_ref.shape[0])


def _node_body(x_ref, ae_ref, at_ref, we2_ref, be2_ref, wt2_ref, bt2_ref,
               wn1x_ref, wn1a_ref, bn1_ref, wn2_ref, bn2_ref, o_ref):
    ae = ae_ref[0] + ae_ref[1]
    at = at_ref[0] + at_ref[1]
    aggr = (jnp.dot(ae[:, :H], we2_ref[...], preferred_element_type=jnp.float32)
            + ae[:, H:H + 1] * be2_ref[...]
            + jnp.dot(at[:, :H], wt2_ref[...], preferred_element_type=jnp.float32)
            + at[:, H:H + 1] * bt2_ref[...])
    xv = x_ref[...]
    h = (jnp.dot(xv, wn1x_ref[...], preferred_element_type=jnp.float32)
         + jnp.dot(aggr, wn1a_ref[...], preferred_element_type=jnp.float32)
         + bn1_ref[...])
    h = h / (1.0 + jnp.exp(-h))
    o_ref[...] = (xv + jnp.dot(h, wn2_ref[...],
                               preferred_element_type=jnp.float32) + bn2_ref[...])


def _make_sc_kernel(n_nodes, n_edges, n_trip):
    ew = n_edges // NW
    mw = n_trip // NW
    ne_ch = ew // C
    nt_ch = mw // C
    n_pad = ((n_nodes + NS * 128 - 1) // (NS * 128)) * (NS * 128)
    rpt = n_pad // NS
    nz = rpt // C

    mesh = plsc.VectorSubcoreMesh(core_axis_name="c", subcore_axis_name="s")

    @functools.partial(
        pl.kernel,
        mesh=mesh,
        compiler_params=pltpu.CompilerParams(use_tc_tiling_on_sc=False),
        out_type=[jax.ShapeDtypeStruct((NC, n_pad, SW), jnp.float32),
                  jax.ShapeDtypeStruct((NC, n_pad, SW), jnp.float32)],
        scratch_types=[
            pltpu.VMEM((8, C), jnp.int32),
            pltpu.VMEM((8, C), jnp.int32),
            pltpu.VMEM((4, C, H), jnp.float32),
            pltpu.VMEM((4, C, H), jnp.float32),
            pltpu.VMEM((5, C, SW), jnp.float32),
            pltpu.VMEM_SHARED((n_pad, SW), jnp.float32),
            pltpu.SemaphoreType.DMA,
            pltpu.SemaphoreType.DMA,
            pltpu.SemaphoreType.DMA,
        ],
    )
    def sc_kernel(row_hbm, col_hbm, ctr_hbm, eproj_hbm, tproj_hbm,
                  xr_hbm, xc_hbm, xt_hbm, agg_e_hbm, agg_t_hbm,
                  ia4, ib4, b1, b2, sbuf, acc, sem_i, sem_g, sem_s):
        cid = lax.axis_index("c")
        sid = lax.axis_index("s")
        wid = sid * NC + cid

        zeros16 = jnp.zeros((LANES,), jnp.float32)

        def zero_sbuf0():
            def zrow(r, carry):
                for cc in range(SW // LANES):
                    sbuf[0, r, pl.ds(cc * LANES, LANES)] = zeros16
                return carry
            lax.fori_loop(0, C, zrow, 0)

        def zero_stripe():
            for k in range(nz):
                pltpu.sync_copy(sbuf.at[0], acc.at[pl.ds(sid * rpt + k * C, C)])

        def flush_stripe(out_hbm):
            for k in range(nz):
                sl = pl.ds(sid * rpt + k * C, C)
                pltpu.sync_copy(acc.at[sl], out_hbm.at[cid, sl])

        def run_phase(nch, base0, idx_hbms, proj_hbm, gat_hbms, out_hbm):
            two = len(idx_hbms) == 2

            def idx_copies(j):
                base = base0 + j * C
                s = lax.rem(j, 8)
                cps = [pltpu.make_async_copy(
                    idx_hbms[0].at[pl.ds(base, C)], ia4.at[s], sem_i)]
                if two:
                    cps.append(pltpu.make_async_copy(
                        idx_hbms[1].at[pl.ds(base, C)], ib4.at[s], sem_i))
                return cps

            def load_copies(j):
                base = base0 + j * C
                s8 = lax.rem(j, 8)
                s5 = lax.rem(j, 5)
                s4 = lax.rem(j, 4)
                cps = [
                    pltpu.make_async_copy(
                        proj_hbm.at[pl.ds(base, C)], sbuf.at[s5], sem_g),
                    pltpu.make_async_copy(
                        gat_hbms[0].at[ia4.at[s8]], b1.at[s4], sem_g),
                ]
                if two:
                    cps.append(pltpu.make_async_copy(
                        gat_hbms[1].at[ib4.at[s8]], b2.at[s4], sem_g))
                return cps

            def scatter_copy(j):
                return pltpu.make_async_copy(
                    sbuf.at[lax.rem(j, 5)], acc.at[ia4.at[lax.rem(j, 8)]],
                    sem_s)

            for k in range(3):
                for cp in idx_copies(k):
                    cp.start()
            for k in range(3):
                for cp in idx_copies(k):
                    cp.wait()
                for cp in load_copies(k):
                    cp.start()
            for cp in idx_copies(3):
                cp.start()

            def body(j, carry):
                @pl.when(j >= 1)
                def _():
                    scatter_copy(j - 1).wait()

                @pl.when(j + 4 < nch)
                def _():
                    for cp in idx_copies(j + 4):
                        cp.start()

                @pl.when(j + 3 < nch)
                def _():
                    for cp in idx_copies(j + 3):
                        cp.wait()
                    for cp in load_copies(j + 3):
                        cp.start()

                for cp in load_copies(j):
                    cp.wait()

                s5 = lax.rem(j, 5)
                s4 = lax.rem(j, 4)

                def crow(r, carry2):
                    for cc in range(H // LANES):
                        sl = pl.ds(cc * LANES, LANES)
                        v = sbuf[s5, r, sl] + b1[s4, r, sl]
                        if two:
                            v = v + b2[s4, r, sl]
                        sbuf[s5, r, sl] = v / (1.0 + jnp.exp(-v))
                    return carry2

                lax.fori_loop(0, C, crow, 0, unroll=4)
                scatter_copy(j).start(add=True)
                return carry

            lax.fori_loop(0, nch, body, 0)
            scatter_copy(nch - 1).wait()
            plsc.subcore_barrier()
            flush_stripe(out_hbm)

        zero_sbuf0()
        zero_stripe()
        plsc.subcore_barrier()
        run_phase(ne_ch, wid * ew, (row_hbm, col_hbm), eproj_hbm,
                  (xr_hbm, xc_hbm), agg_e_hbm)
        zero_sbuf0()
        zero_stripe()
        plsc.subcore_barrier()
        run_phase(nt_ch, wid * mw, (ctr_hbm,), tproj_hbm,
                  (xt_hbm,), agg_t_hbm)

    return sc_kernel


def kernel(x, edge_index, edge_attr_rbf, triplet_index, angles,
           W_e1, b_e1, W_e2, b_e2,
           W_t1, b_t1, W_t2, b_t2,
           W_n1, b_n1, W_n2, b_n2,
           centers):
    n_nodes, h = x.shape
    n_edges = edge_index.shape[1]
    n_trip = triplet_index.shape[0]
    n_rbf_a = centers.shape[0]
    sigma = math.pi / n_rbf_a
    inv_sig2 = 1.0 / (sigma * sigma)

    We1a, We1b, We1c = W_e1[:h], W_e1[h:2 * h], W_e1[2 * h:]
    Wt1a, Wt1b = W_t1[:h], W_t1[h:]
    Wn1x, Wn1a = W_n1[:h], W_n1[h:]
    be1 = b_e1.reshape(1, h)
    bt1 = b_t1.reshape(1, h)
    be2 = b_e2.reshape(1, h)
    bt2 = b_t2.reshape(1, h)
    bn1 = b_n1.reshape(1, h)
    bn2 = b_n2.reshape(1, h)
    row1 = edge_index[0]
    col1 = edge_index[1]
    ctr1 = triplet_index[:, 1]
    ang2 = angles.reshape(n_trip, 1)
    cen2 = centers.reshape(1, n_rbf_a)

    xr, xc, xt = pl.pallas_call(
        _proj3_body,
        out_shape=[jax.ShapeDtypeStruct((n_nodes, h), jnp.float32)] * 3,
    )(x, We1a, We1b, Wt1a)

    BE = 2000
    eproj = pl.pallas_call(
        _eproj_body,
        grid=(n_edges // BE,),
        in_specs=[pl.BlockSpec((BE, NRBF), lambda i: (i, 0)),
                  pl.BlockSpec((NRBF, h), lambda i: (0, 0)),
                  pl.BlockSpec((1, h), lambda i: (0, 0))],
        out_specs=pl.BlockSpec((BE, SW), lambda i: (i, 0)),
        out_shape=jax.ShapeDtypeStruct((n_edges, SW), jnp.float32),
    )(edge_attr_rbf, We1c, be1)

    BT = 2560
    tproj = pl.pallas_call(
        functools.partial(_tproj_body, inv_sig2=inv_sig2),
        grid=(n_trip // BT,),
        in_specs=[pl.BlockSpec((BT, 1), lambda i: (i, 0)),
                  pl.BlockSpec((1, n_rbf_a), lambda i: (0, 0)),
                  pl.BlockSpec((n_rbf_a, h), lambda i: (0, 0)),
                  pl.BlockSpec((1, h), lambda i: (0, 0))],
        out_specs=pl.BlockSpec((BT, SW), lambda i: (i, 0)),
        out_shape=jax.ShapeDtypeStruct((n_trip, SW), jnp.float32),
    )(ang2, cen2, Wt1b, bt1)

    sc = _make_sc_kernel(n_nodes, n_edges, n_trip)
    agg_e, agg_t = sc(row1, col1, ctr1, eproj, tproj, xr, xc, xt)

    BN = 2000
    out = pl.pallas_call(
        _node_body,
        grid=(n_nodes // BN,),
        in_specs=[pl.BlockSpec((BN, h), lambda i: (i, 0)),
                  pl.BlockSpec((NC, BN, SW), lambda i: (0, i, 0)),
                  pl.BlockSpec((NC, BN, SW), lambda i: (0, i, 0)),
                  pl.BlockSpec((h, h), lambda i: (0, 0)),
                  pl.BlockSpec((1, h), lambda i: (0, 0)),
                  pl.BlockSpec((h, h), lambda i: (0, 0)),
                  pl.BlockSpec((1, h), lambda i: (0, 0)),
                  pl.BlockSpec((h, h), lambda i: (0, 0)),
                  pl.BlockSpec((h, h), lambda i: (0, 0)),
                  pl.BlockSpec((1, h), lambda i: (0, 0)),
                  pl.BlockSpec((h, h), lambda i: (0, 0)),
                  pl.BlockSpec((1, h), lambda i: (0, 0))],
        out_specs=pl.BlockSpec((BN, h), lambda i: (i, 0)),
        out_shape=jax.ShapeDtypeStruct((n_nodes, h), jnp.float32),
    )(x, agg_e, agg_t, W_e2, be2, W_t2, bt2, Wn1x, Wn1a, bn1, W_n2, bn2)
    return out


# speedup vs baseline: 1.0482x; 1.0482x over previous
import functools
import math

import jax
import jax.numpy as jnp
from jax import lax
from jax.experimental import pallas as pl
from jax.experimental.pallas import tpu as pltpu
from jax.experimental.pallas import tpu_sc as plsc

H = 128
NRBF = 32
SW = 144
NC, NS, LANES = 2, 16, 16
NW = NC * NS
C = 40


def _proj3_body(x_ref, wa_ref, wb_ref, wc_ref, oa_ref, ob_ref, oc_ref):
    xv = x_ref[...]
    oa_ref[...] = jnp.dot(xv, wa_ref[...], preferred_element_type=jnp.float32)
    ob_ref[...] = jnp.dot(xv, wb_ref[...], preferred_element_type=jnp.float32)
    oc_ref[...] = jnp.dot(xv, wc_ref[...], preferred_element_type=jnp.float32)


def _count_col(rows):
    lane = lax.broadcasted_iota(jnp.int32, (rows, SW - H), 1)
    return jnp.where(lane == 0, 1.0, 0.0).astype(jnp.float32)


def _eproj_body(a_ref, w_ref, b_ref, o_ref):
    o_ref[:, :H] = (jnp.dot(a_ref[...], w_ref[...],
                            preferred_element_type=jnp.float32) + b_ref[...])
    o_ref[:, H:] = _count_col(o_ref.shape[0])


def _tproj_body(ang_ref, cen_ref, w_ref, b_ref, o_ref, *, inv_sig2):
    d = ang_ref[...] - cen_ref[...]
    rbf = jnp.exp(-(d * d) * inv_sig2)
    o_ref[:, :H] = (jnp.dot(rbf, w_ref[...],
                            preferred_element_type=jnp.float32) + b_ref[...])
    o_ref[:, H:] = _count_col(o_ref.shape[0])


def _node_body(x_ref, ae_ref, at_ref, we2_ref, be2_ref, wt2_ref, bt2_ref,
               wn1x_ref, wn1a_ref, bn1_ref, wn2_ref, bn2_ref, o_ref):
    ae = ae_ref[0] + ae_ref[1]
    at = at_ref[0] + at_ref[1]
    aggr = (jnp.dot(ae[:, :H], we2_ref[...], preferred_element_type=jnp.float32)
            + ae[:, H:H + 1] * be2_ref[...]
            + jnp.dot(at[:, :H], wt2_ref[...], preferred_element_type=jnp.float32)
            + at[:, H:H + 1] * bt2_ref[...])
    xv = x_ref[...]
    h = (jnp.dot(xv, wn1x_ref[...], preferred_element_type=jnp.float32)
         + jnp.dot(aggr, wn1a_ref[...], preferred_element_type=jnp.float32)
         + bn1_ref[...])
    h = h / (1.0 + jnp.exp(-h))
    o_ref[...] = (xv + jnp.dot(h, wn2_ref[...],
                               preferred_element_type=jnp.float32) + bn2_ref[...])


def _make_sc_kernel(n_nodes, n_edges, n_trip):
    ew = n_edges // NW
    mw = n_trip // NW
    ne_ch = ew // C
    nt_ch = mw // C
    n_pad = ((n_nodes + NS * 128 - 1) // (NS * 128)) * (NS * 128)
    rpt = n_pad // NS
    nz = rpt // C

    mesh = plsc.VectorSubcoreMesh(core_axis_name="c", subcore_axis_name="s")

    @functools.partial(
        pl.kernel,
        mesh=mesh,
        compiler_params=pltpu.CompilerParams(use_tc_tiling_on_sc=False),
        out_type=[jax.ShapeDtypeStruct((NC, n_pad, SW), jnp.float32),
                  jax.ShapeDtypeStruct((NC, n_pad, SW), jnp.float32)],
        scratch_types=[
            pltpu.VMEM((4, C), jnp.int32),
            pltpu.VMEM((4, C), jnp.int32),
            pltpu.VMEM((2, C, H), jnp.float32),
            pltpu.VMEM((2, C, H), jnp.float32),
            pltpu.VMEM((3, C, SW), jnp.float32),
            pltpu.VMEM_SHARED((n_pad, SW), jnp.float32),
            pltpu.SemaphoreType.DMA,
            pltpu.SemaphoreType.DMA,
            pltpu.SemaphoreType.DMA,
        ],
    )
    def sc_kernel(row_hbm, col_hbm, ctr_hbm, eproj_hbm, tproj_hbm,
                  xr_hbm, xc_hbm, xt_hbm, zeros_hbm, agg_e_hbm, agg_t_hbm,
                  ia4, ib4, b1, b2, sbuf, acc, sem_i, sem_g, sem_s):
        cid = lax.axis_index("c")
        sid = lax.axis_index("s")
        wid = sid * NC + cid

        def zero_stripe():
            pltpu.sync_copy(zeros_hbm, acc.at[pl.ds(sid * rpt, rpt)])

        def flush_stripe(out_hbm):
            sl = pl.ds(sid * rpt, rpt)
            pltpu.sync_copy(acc.at[sl], out_hbm.at[cid, sl])

        def run_phase(nch, base0, idx_hbms, proj_hbm, gat_hbms, out_hbm):
            two = len(idx_hbms) == 2

            def idx_copies(j):
                base = base0 + j * C
                s = lax.rem(j, 4)
                cps = [pltpu.make_async_copy(
                    idx_hbms[0].at[pl.ds(base, C)], ia4.at[s], sem_i)]
                if two:
                    cps.append(pltpu.make_async_copy(
                        idx_hbms[1].at[pl.ds(base, C)], ib4.at[s], sem_i))
                return cps

            def load_copies(j):
                base = base0 + j * C
                s4 = lax.rem(j, 4)
                s3 = lax.rem(j, 3)
                s2 = lax.rem(j, 2)
                cps = [
                    pltpu.make_async_copy(
                        proj_hbm.at[pl.ds(base, C)], sbuf.at[s3], sem_g),
                    pltpu.make_async_copy(
                        gat_hbms[0].at[ia4.at[s4]], b1.at[s2], sem_g),
                ]
                if two:
                    cps.append(pltpu.make_async_copy(
                        gat_hbms[1].at[ib4.at[s4]], b2.at[s2], sem_g))
                return cps

            def scatter_copy(j):
                return pltpu.make_async_copy(
                    sbuf.at[lax.rem(j, 3)], acc.at[ia4.at[lax.rem(j, 4)]],
                    sem_s)

            for cp in idx_copies(0):
                cp.start()
            for cp in idx_copies(0):
                cp.wait()
            for cp in idx_copies(1):
                cp.start()
            for cp in load_copies(0):
                cp.start()

            def body(j, carry):
                @pl.when(j >= 2)
                def _():
                    scatter_copy(j - 2).wait()

                @pl.when(j + 2 < nch)
                def _():
                    for cp in idx_copies(j + 2):
                        cp.start()

                @pl.when(j + 1 < nch)
                def _():
                    for cp in idx_copies(j + 1):
                        cp.wait()
                    for cp in load_copies(j + 1):
                        cp.start()

                for cp in load_copies(j):
                    cp.wait()

                s3 = lax.rem(j, 3)
                s2 = lax.rem(j, 2)

                def crow(r, carry2):
                    for cc in range(H // LANES):
                        sl = pl.ds(cc * LANES, LANES)
                        v = sbuf[s3, r, sl] + b1[s2, r, sl]
                        if two:
                            v = v + b2[s2, r, sl]
                        sbuf[s3, r, sl] = v / (1.0 + jnp.exp(-v))
                    return carry2

                lax.fori_loop(0, C, crow, 0, unroll=4)
                scatter_copy(j).start(add=True)
                return carry

            lax.fori_loop(0, nch, body, 0)
            scatter_copy(nch - 2).wait()
            scatter_copy(nch - 1).wait()
            plsc.subcore_barrier()
            flush_stripe(out_hbm)

        zero_stripe()
        plsc.subcore_barrier()
        run_phase(ne_ch, wid * ew, (row_hbm, col_hbm), eproj_hbm,
                  (xr_hbm, xc_hbm), agg_e_hbm)
        zero_stripe()
        plsc.subcore_barrier()
        run_phase(nt_ch, wid * mw, (ctr_hbm,), tproj_hbm,
                  (xt_hbm,), agg_t_hbm)

    return sc_kernel


def kernel(x, edge_index, edge_attr_rbf, triplet_index, angles,
           W_e1, b_e1, W_e2, b_e2,
           W_t1, b_t1, W_t2, b_t2,
           W_n1, b_n1, W_n2, b_n2,
           centers):
    n_nodes, h = x.shape
    n_edges = edge_index.shape[1]
    n_trip = triplet_index.shape[0]
    n_rbf_a = centers.shape[0]
    sigma = math.pi / n_rbf_a
    inv_sig2 = 1.0 / (sigma * sigma)

    We1a, We1b, We1c = W_e1[:h], W_e1[h:2 * h], W_e1[2 * h:]
    Wt1a, Wt1b = W_t1[:h], W_t1[h:]
    Wn1x, Wn1a = W_n1[:h], W_n1[h:]
    be1 = b_e1.reshape(1, h)
    bt1 = b_t1.reshape(1, h)
    be2 = b_e2.reshape(1, h)
    bt2 = b_t2.reshape(1, h)
    bn1 = b_n1.reshape(1, h)
    bn2 = b_n2.reshape(1, h)
    row1 = edge_index[0]
    col1 = edge_index[1]
    ctr1 = triplet_index[:, 1]
    ang2 = angles.reshape(n_trip, 1)
    cen2 = centers.reshape(1, n_rbf_a)

    xr, xc, xt = pl.pallas_call(
        _proj3_body,
        out_shape=[jax.ShapeDtypeStruct((n_nodes, h), jnp.float32)] * 3,
    )(x, We1a, We1b, Wt1a)

    BE = 2000
    eproj = pl.pallas_call(
        _eproj_body,
        grid=(n_edges // BE,),
        in_specs=[pl.BlockSpec((BE, NRBF), lambda i: (i, 0)),
                  pl.BlockSpec((NRBF, h), lambda i: (0, 0)),
                  pl.BlockSpec((1, h), lambda i: (0, 0))],
        out_specs=pl.BlockSpec((BE, SW), lambda i: (i, 0)),
        out_shape=jax.ShapeDtypeStruct((n_edges, SW), jnp.float32),
    )(edge_attr_rbf, We1c, be1)

    BT = 2560
    tproj = pl.pallas_call(
        functools.partial(_tproj_body, inv_sig2=inv_sig2),
        grid=(n_trip // BT,),
        in_specs=[pl.BlockSpec((BT, 1), lambda i: (i, 0)),
                  pl.BlockSpec((1, n_rbf_a), lambda i: (0, 0)),
                  pl.BlockSpec((n_rbf_a, h), lambda i: (0, 0)),
                  pl.BlockSpec((1, h), lambda i: (0, 0))],
        out_specs=pl.BlockSpec((BT, SW), lambda i: (i, 0)),
        out_shape=jax.ShapeDtypeStruct((n_trip, SW), jnp.float32),
    )(ang2, cen2, Wt1b, bt1)

    sc = _make_sc_kernel(n_nodes, n_edges, n_trip)
    zeros_blk = jnp.zeros((((n_nodes + NS * 128 - 1) // (NS * 128)) * 128, SW),
                          jnp.float32)
    agg_e, agg_t = sc(row1, col1, ctr1, eproj, tproj, xr, xc, xt, zeros_blk)

    BN = 2000
    out = pl.pallas_call(
        _node_body,
        grid=(n_nodes // BN,),
        in_specs=[pl.BlockSpec((BN, h), lambda i: (i, 0)),
                  pl.BlockSpec((NC, BN, SW), lambda i: (0, i, 0)),
                  pl.BlockSpec((NC, BN, SW), lambda i: (0, i, 0)),
                  pl.BlockSpec((h, h), lambda i: (0, 0)),
                  pl.BlockSpec((1, h), lambda i: (0, 0)),
                  pl.BlockSpec((h, h), lambda i: (0, 0)),
                  pl.BlockSpec((1, h), lambda i: (0, 0)),
                  pl.BlockSpec((h, h), lambda i: (0, 0)),
                  pl.BlockSpec((h, h), lambda i: (0, 0)),
                  pl.BlockSpec((1, h), lambda i: (0, 0)),
                  pl.BlockSpec((h, h), lambda i: (0, 0)),
                  pl.BlockSpec((1, h), lambda i: (0, 0))],
        out_specs=pl.BlockSpec((BN, h), lambda i: (i, 0)),
        out_shape=jax.ShapeDtypeStruct((n_nodes, h), jnp.float32),
    )(x, agg_e, agg_t, W_e2, be2, W_t2, bt2, Wn1x, Wn1a, bn1, W_n2, bn2)
    return out


# speedup vs baseline: 1.1398x; 1.0874x over previous
import functools
import math

import jax
import jax.numpy as jnp
from jax import lax
from jax.experimental import pallas as pl
from jax.experimental.pallas import tpu as pltpu
from jax.experimental.pallas import tpu_sc as plsc

H = 128
NRBF = 32
SW = 144
NC, NS, LANES = 2, 16, 16
NW = NC * NS
C = 40


def _proj3_body(x_ref, wa_ref, wb_ref, wc_ref, oa_ref, ob_ref, oc_ref):
    xv = x_ref[...]
    f = jnp.float32
    oa_ref[...] = jnp.dot(xv, wa_ref[...], preferred_element_type=f).astype(jnp.bfloat16)
    ob_ref[...] = jnp.dot(xv, wb_ref[...], preferred_element_type=f).astype(jnp.bfloat16)
    oc_ref[...] = jnp.dot(xv, wc_ref[...], preferred_element_type=f).astype(jnp.bfloat16)


def _perm128():
    p = []
    for b in range(4):
        for i in range(16):
            p.extend((32 * b + i, 32 * b + 16 + i))
    return p


def _count_col(rows):
    lane = lax.broadcasted_iota(jnp.int32, (rows, SW - H), 1)
    return jnp.where(lane == 0, 1.0, 0.0).astype(jnp.float32)


def _eproj_body(a_ref, w_ref, b_ref, o_ref):
    o_ref[:, :H] = (jnp.dot(a_ref[...], w_ref[...],
                            preferred_element_type=jnp.float32) + b_ref[...])
    o_ref[:, H:] = _count_col(o_ref.shape[0])


def _tproj_body(ang_ref, cen_ref, w_ref, b_ref, o_ref, *, inv_sig2):
    d = ang_ref[...] - cen_ref[...]
    rbf = jnp.exp(-(d * d) * inv_sig2)
    o_ref[:, :H] = (jnp.dot(rbf, w_ref[...],
                            preferred_element_type=jnp.float32) + b_ref[...])
    o_ref[:, H:] = _count_col(o_ref.shape[0])


def _node_body(x_ref, ae_ref, at_ref, we2_ref, be2_ref, wt2_ref, bt2_ref,
               wn1x_ref, wn1a_ref, bn1_ref, wn2_ref, bn2_ref, o_ref):
    ae = ae_ref[0] + ae_ref[1]
    at = at_ref[0] + at_ref[1]
    aggr = (jnp.dot(ae[:, :H], we2_ref[...], preferred_element_type=jnp.float32)
            + ae[:, H:H + 1] * be2_ref[...]
            + jnp.dot(at[:, :H], wt2_ref[...], preferred_element_type=jnp.float32)
            + at[:, H:H + 1] * bt2_ref[...])
    xv = x_ref[...]
    h = (jnp.dot(xv, wn1x_ref[...], preferred_element_type=jnp.float32)
         + jnp.dot(aggr, wn1a_ref[...], preferred_element_type=jnp.float32)
         + bn1_ref[...])
    h = h / (1.0 + jnp.exp(-h))
    o_ref[...] = (xv + jnp.dot(h, wn2_ref[...],
                               preferred_element_type=jnp.float32) + bn2_ref[...])


def _make_sc_kernel(n_nodes, n_edges, n_trip):
    ew = n_edges // NW
    mw = n_trip // NW
    ne_ch = ew // C
    nt_ch = mw // C
    n_pad = ((n_nodes + NS * 128 - 1) // (NS * 128)) * (NS * 128)
    rpt = n_pad // NS
    nz = rpt // C

    mesh = plsc.VectorSubcoreMesh(core_axis_name="c", subcore_axis_name="s")

    @functools.partial(
        pl.kernel,
        mesh=mesh,
        compiler_params=pltpu.CompilerParams(use_tc_tiling_on_sc=False, needs_layout_passes=False),
        out_type=[jax.ShapeDtypeStruct((NC, n_pad, SW), jnp.float32),
                  jax.ShapeDtypeStruct((NC, n_pad, SW), jnp.float32)],
        scratch_types=[
            pltpu.VMEM((4, C), jnp.int32),
            pltpu.VMEM((4, C), jnp.int32),
            pltpu.VMEM((2, C, H // 2), jnp.int32),
            pltpu.VMEM((2, C, H // 2), jnp.int32),
            pltpu.VMEM((3, C, SW), jnp.float32),
            pltpu.VMEM_SHARED((n_pad, SW), jnp.float32),
            pltpu.SemaphoreType.DMA,
            pltpu.SemaphoreType.DMA,
            pltpu.SemaphoreType.DMA,
        ],
    )
    def sc_kernel(row_hbm, col_hbm, ctr_hbm, eproj_hbm, tproj_hbm,
                  xr_hbm, xc_hbm, xt_hbm, zeros_hbm, agg_e_hbm, agg_t_hbm,
                  ia4, ib4, b1, b2, sbuf, acc, sem_i, sem_g, sem_s):
        cid = lax.axis_index("c")
        sid = lax.axis_index("s")
        wid = sid * NC + cid

        def zero_stripe():
            pltpu.sync_copy(zeros_hbm, acc.at[pl.ds(sid * rpt, rpt)])

        def flush_stripe(out_hbm):
            sl = pl.ds(sid * rpt, rpt)
            pltpu.sync_copy(acc.at[sl], out_hbm.at[cid, sl])

        def run_phase(nch, base0, idx_hbms, proj_hbm, gat_hbms, out_hbm):
            two = len(idx_hbms) == 2

            def idx_copies(j):
                base = base0 + j * C
                s = lax.rem(j, 4)
                cps = [pltpu.make_async_copy(
                    idx_hbms[0].at[pl.ds(base, C)], ia4.at[s], sem_i)]
                if two:
                    cps.append(pltpu.make_async_copy(
                        idx_hbms[1].at[pl.ds(base, C)], ib4.at[s], sem_i))
                return cps

            def load_copies(j):
                base = base0 + j * C
                s4 = lax.rem(j, 4)
                s3 = lax.rem(j, 3)
                s2 = lax.rem(j, 2)
                cps = [
                    pltpu.make_async_copy(
                        proj_hbm.at[pl.ds(base, C)], sbuf.at[s3], sem_g),
                    pltpu.make_async_copy(
                        gat_hbms[0].at[ia4.at[s4]], b1.at[s2], sem_g),
                ]
                if two:
                    cps.append(pltpu.make_async_copy(
                        gat_hbms[1].at[ib4.at[s4]], b2.at[s2], sem_g))
                return cps

            def scatter_copy(j):
                return pltpu.make_async_copy(
                    sbuf.at[lax.rem(j, 3)], acc.at[ia4.at[lax.rem(j, 4)]],
                    sem_s)

            for cp in idx_copies(0):
                cp.start()
            for cp in idx_copies(0):
                cp.wait()
            for cp in idx_copies(1):
                cp.start()
            for cp in load_copies(0):
                cp.start()

            def body(j, carry):
                @pl.when(j >= 2)
                def _():
                    scatter_copy(j - 2).wait()

                @pl.when(j + 2 < nch)
                def _():
                    for cp in idx_copies(j + 2):
                        cp.start()

                @pl.when(j + 1 < nch)
                def _():
                    for cp in idx_copies(j + 1):
                        cp.wait()
                    for cp in load_copies(j + 1):
                        cp.start()

                for cp in load_copies(j):
                    cp.wait()

                s3 = lax.rem(j, 3)
                s2 = lax.rem(j, 2)

                himask = jnp.full((LANES,), -65536, jnp.int32)

                def crow(r, carry2):
                    for g in range(H // (2 * LANES)):
                        wsl = pl.ds(g * LANES, LANES)
                        w1 = b1[s2, r, wsl]
                        lo = plsc.bitcast(lax.shift_left(w1, 16), jnp.float32)
                        hi = plsc.bitcast(w1 & himask, jnp.float32)
                        if two:
                            w2 = b2[s2, r, wsl]
                            lo = lo + plsc.bitcast(
                                lax.shift_left(w2, 16), jnp.float32)
                            hi = hi + plsc.bitcast(w2 & himask, jnp.float32)
                        sl_lo = pl.ds(2 * g * LANES, LANES)
                        sl_hi = pl.ds((2 * g + 1) * LANES, LANES)
                        v = sbuf[s3, r, sl_lo] + lo
                        sbuf[s3, r, sl_lo] = v / (1.0 + jnp.exp(-v))
                        v = sbuf[s3, r, sl_hi] + hi
                        sbuf[s3, r, sl_hi] = v / (1.0 + jnp.exp(-v))
                    return carry2

                lax.fori_loop(0, C, crow, 0, unroll=4)
                scatter_copy(j).start(add=True)
                return carry

            lax.fori_loop(0, nch, body, 0)
            scatter_copy(nch - 2).wait()
            scatter_copy(nch - 1).wait()
            plsc.subcore_barrier()
            flush_stripe(out_hbm)

        zero_stripe()
        plsc.subcore_barrier()
        run_phase(ne_ch, wid * ew, (row_hbm, col_hbm), eproj_hbm,
                  (xr_hbm, xc_hbm), agg_e_hbm)
        zero_stripe()
        plsc.subcore_barrier()
        run_phase(nt_ch, wid * mw, (ctr_hbm,), tproj_hbm,
                  (xt_hbm,), agg_t_hbm)

    return sc_kernel


def kernel(x, edge_index, edge_attr_rbf, triplet_index, angles,
           W_e1, b_e1, W_e2, b_e2,
           W_t1, b_t1, W_t2, b_t2,
           W_n1, b_n1, W_n2, b_n2,
           centers):
    n_nodes, h = x.shape
    n_edges = edge_index.shape[1]
    n_trip = triplet_index.shape[0]
    n_rbf_a = centers.shape[0]
    sigma = math.pi / n_rbf_a
    inv_sig2 = 1.0 / (sigma * sigma)

    We1a, We1b, We1c = W_e1[:h], W_e1[h:2 * h], W_e1[2 * h:]
    Wt1a, Wt1b = W_t1[:h], W_t1[h:]
    Wn1x, Wn1a = W_n1[:h], W_n1[h:]
    be1 = b_e1.reshape(1, h)
    bt1 = b_t1.reshape(1, h)
    be2 = b_e2.reshape(1, h)
    bt2 = b_t2.reshape(1, h)
    bn1 = b_n1.reshape(1, h)
    bn2 = b_n2.reshape(1, h)
    row1 = edge_index[0]
    col1 = edge_index[1]
    ctr1 = triplet_index[:, 1]
    ang2 = angles.reshape(n_trip, 1)
    cen2 = centers.reshape(1, n_rbf_a)

    perm = jnp.array(_perm128(), jnp.int32)
    xr, xc, xt = pl.pallas_call(
        _proj3_body,
        out_shape=[jax.ShapeDtypeStruct((n_nodes, h), jnp.bfloat16)] * 3,
    )(x, We1a[:, perm], We1b[:, perm], Wt1a[:, perm])

    def _pack_i32(t):
        return jax.lax.bitcast_convert_type(
            t.reshape(n_nodes, h // 2, 2), jnp.int32)

    xr, xc, xt = _pack_i32(xr), _pack_i32(xc), _pack_i32(xt)

    BE = 2000
    eproj = pl.pallas_call(
        _eproj_body,
        grid=(n_edges // BE,),
        in_specs=[pl.BlockSpec((BE, NRBF), lambda i: (i, 0)),
                  pl.BlockSpec((NRBF, h), lambda i: (0, 0)),
                  pl.BlockSpec((1, h), lambda i: (0, 0))],
        out_specs=pl.BlockSpec((BE, SW), lambda i: (i, 0)),
        out_shape=jax.ShapeDtypeStruct((n_edges, SW), jnp.float32),
    )(edge_attr_rbf, We1c, be1)

    BT = 2560
    tproj = pl.pallas_call(
        functools.partial(_tproj_body, inv_sig2=inv_sig2),
        grid=(n_trip // BT,),
        in_specs=[pl.BlockSpec((BT, 1), lambda i: (i, 0)),
                  pl.BlockSpec((1, n_rbf_a), lambda i: (0, 0)),
                  pl.BlockSpec((n_rbf_a, h), lambda i: (0, 0)),
                  pl.BlockSpec((1, h), lambda i: (0, 0))],
        out_specs=pl.BlockSpec((BT, SW), lambda i: (i, 0)),
        out_shape=jax.ShapeDtypeStruct((n_trip, SW), jnp.float32),
    )(ang2, cen2, Wt1b, bt1)

    sc = _make_sc_kernel(n_nodes, n_edges, n_trip)
    zeros_blk = jnp.zeros((((n_nodes + NS * 128 - 1) // (NS * 128)) * 128, SW),
                          jnp.float32)
    agg_e, agg_t = sc(row1, col1, ctr1, eproj, tproj, xr, xc, xt, zeros_blk)

    BN = 2000
    out = pl.pallas_call(
        _node_body,
        grid=(n_nodes // BN,),
        in_specs=[pl.BlockSpec((BN, h), lambda i: (i, 0)),
                  pl.BlockSpec((NC, BN, SW), lambda i: (0, i, 0)),
                  pl.BlockSpec((NC, BN, SW), lambda i: (0, i, 0)),
                  pl.BlockSpec((h, h), lambda i: (0, 0)),
                  pl.BlockSpec((1, h), lambda i: (0, 0)),
                  pl.BlockSpec((h, h), lambda i: (0, 0)),
                  pl.BlockSpec((1, h), lambda i: (0, 0)),
                  pl.BlockSpec((h, h), lambda i: (0, 0)),
                  pl.BlockSpec((h, h), lambda i: (0, 0)),
                  pl.BlockSpec((1, h), lambda i: (0, 0)),
                  pl.BlockSpec((h, h), lambda i: (0, 0)),
                  pl.BlockSpec((1, h), lambda i: (0, 0))],
        out_specs=pl.BlockSpec((BN, h), lambda i: (i, 0)),
        out_shape=jax.ShapeDtypeStruct((n_nodes, h), jnp.float32),
    )(x, agg_e, agg_t, W_e2, be2, W_t2, bt2, Wn1x, Wn1a, bn1, W_n2, bn2)
    return out
